# baseline (speedup 1.0000x reference)
import functools

import jax
import jax.numpy as jnp
from jax import lax
from jax.experimental import pallas as pl
from jax.experimental.pallas import tpu as pltpu
from jax.experimental.pallas import tpu_sc as plsc

N = 100000
NPAD = 100096
ROWS_PER_TILE = 6256
TRASH = 100000
J2 = 2
G = 384
B = 2000

CHUNKS_T = 10
CHUNKS_I = 124


_MESH = plsc.VectorSubcoreMesh(core_axis_name="c", subcore_axis_name="s")


def _gather_waits(table, src_v, rows_v, sem_g):
    for j in range(J2):
        pltpu.make_async_copy(table.at[src_v.at[j]], rows_v.at[j], sem_g).wait()


def _gather_fires(table, src_v, rows_v, sem_g):
    for j in range(J2):
        pltpu.async_copy(table.at[src_v.at[j]], rows_v.at[j], sem_g)


def _scatter_waits(rows_v, acc, dst_v, sem_s):
    for j in range(J2):
        pltpu.make_async_copy(rows_v.at[j], acc.at[dst_v.at[j]], sem_s).wait()


def _scatter_fires(rows_v, acc, dst_v, sem_s):
    for j in range(J2):
        pltpu.async_copy(rows_v.at[j], acc.at[dst_v.at[j]], sem_s, add=True)


def _fill_trash(dst_v):
    tr = jnp.full((16,), TRASH, jnp.int32)
    for j in range(J2):
        for i in range(G // 16):
            dst_v[j, pl.ds(i * 16, 16)] = tr


def _make_segsum(ch):
    @functools.partial(
        pl.kernel,
        out_type=jax.ShapeDtypeStruct((2, NPAD, 16), jnp.float32),
        mesh=_MESH,
        scratch_types=[
            pltpu.VMEM((J2, G), jnp.int32),
            pltpu.VMEM((J2, G), jnp.int32),
            pltpu.VMEM((J2, G, 16), jnp.float32),
            pltpu.VMEM((J2, G), jnp.int32),
            pltpu.VMEM((J2, G), jnp.int32),
            pltpu.VMEM((J2, G, 16), jnp.float32),
            pltpu.VMEM_SHARED((NPAD, 16), jnp.float32),
            pltpu.SemaphoreType.DMA,
            pltpu.SemaphoreType.DMA,
        ],
        compiler_params=pltpu.CompilerParams(use_tc_tiling_on_sc=False),
    )
    def seg(table, src2, dstg, zeros, out,
            src0, dst0, rows0, src1, dst1, rows1, acc, sem_g, sem_s):
        c = lax.axis_index("c")
        s = lax.axis_index("s")
        pltpu.sync_copy(zeros, acc.at[pl.ds(s * ROWS_PER_TILE, ROWS_PER_TILE)])
        _fill_trash(dst1)
        plsc.subcore_barrier()
        _scatter_fires(rows1, acc, dst1, sem_s)
        base0 = s * ch * J2
        pltpu.sync_copy(src2.at[c].at[pl.ds(base0, J2)], src0)
        pltpu.sync_copy(dstg.at[pl.ds(base0, J2)], dst0)
        _gather_fires(table, src0, rows0, sem_g)

        def half(k, bsrc, bdst, brows, osrc, odst, orows, nxt):
            _gather_waits(table, bsrc, brows, sem_g)
            _scatter_waits(orows, acc, odst, sem_s)
            _scatter_fires(brows, acc, bdst, sem_s)
            base = (s * ch + nxt) * J2
            pltpu.sync_copy(src2.at[c].at[pl.ds(base, J2)], osrc)
            pltpu.sync_copy(dstg.at[pl.ds(base, J2)], odst)
            _gather_fires(table, osrc, orows, sem_g)

        def pair(k, carry):
            half(k, src0, dst0, rows0, src1, dst1, rows1, 2 * k + 1)
            half(k, src1, dst1, rows1, src0, dst0, rows0, 2 * k + 2)
            return carry

        lax.fori_loop(0, ch // 2, pair, 0)
        _gather_waits(table, src0, rows0, sem_g)
        _scatter_waits(rows1, acc, dst1, sem_s)
        plsc.subcore_barrier()
        pltpu.sync_copy(
            acc.at[pl.ds(s * ROWS_PER_TILE, ROWS_PER_TILE)],
            out.at[c].at[pl.ds(s * ROWS_PER_TILE, ROWS_PER_TILE)],
        )

    return seg


_seg_t = _make_segsum(CHUNKS_T)
_seg_i = _make_segsum(CHUNKS_I)


def _make_cnt(ch):
    @functools.partial(
        pl.kernel,
        out_type=jax.ShapeDtypeStruct((2, NPAD, 16), jnp.float32),
        mesh=_MESH,
        scratch_types=[
            pltpu.VMEM((J2, G), jnp.int32),
            pltpu.VMEM((J2, G), jnp.int32),
            pltpu.VMEM((J2, G, 16), jnp.float32),
            pltpu.VMEM_SHARED((NPAD, 16), jnp.float32),
            pltpu.SemaphoreType.DMA,
        ],
        compiler_params=pltpu.CompilerParams(use_tc_tiling_on_sc=False),
    )
    def cnt(dstg, zeros, ones, out, dst0, dst1, rows1, acc, sem_s):
        c = lax.axis_index("c")
        s = lax.axis_index("s")
        pltpu.sync_copy(zeros, acc.at[pl.ds(s * ROWS_PER_TILE, ROWS_PER_TILE)])
        pltpu.sync_copy(ones, rows1)
        _fill_trash(dst1)
        plsc.subcore_barrier()
        _scatter_fires(rows1, acc, dst1, sem_s)
        base0 = s * ch * J2
        pltpu.sync_copy(dstg.at[pl.ds(base0, J2)], dst0)

        def half(bdst, odst, nxt):
            _scatter_waits(rows1, acc, odst, sem_s)
            _scatter_fires(rows1, acc, bdst, sem_s)
            base = (s * ch + nxt) * J2
            pltpu.sync_copy(dstg.at[pl.ds(base, J2)], odst)

        def pair(k, carry):
            half(dst0, dst1, 2 * k + 1)
            half(dst1, dst0, 2 * k + 2)
            return carry

        lax.fori_loop(0, ch // 2, pair, 0)
        _scatter_waits(rows1, acc, dst1, sem_s)
        plsc.subcore_barrier()
        pltpu.sync_copy(
            acc.at[pl.ds(s * ROWS_PER_TILE, ROWS_PER_TILE)],
            out.at[c].at[pl.ds(s * ROWS_PER_TILE, ROWS_PER_TILE)],
        )

    return cnt


_cnt_i = _make_cnt(CHUNKS_I)


def _prep_edges(src, dst, ngroups):
    e = src.shape[0]
    epad = (ngroups + J2) * G
    srcp = jnp.concatenate([src, jnp.zeros((epad - e,), jnp.int32)])
    dstp = jnp.concatenate([dst, jnp.full((epad - e,), TRASH, jnp.int32)])
    src2 = jnp.stack([srcp, srcp + N]).reshape(2, ngroups + J2, G)
    return src2, dstp.reshape(ngroups + J2, G)


def _pre0_body(x_ref, wt_ref, wi_ref, ot_ref, oi_ref):
    xb = x_ref[...]
    ht = jnp.dot(xb, wt_ref[...], preferred_element_type=jnp.float32)
    hi = jnp.dot(xb, wi_ref[...], preferred_element_type=jnp.float32)
    ot_ref[0] = ht[:, :16]
    ot_ref[1] = ht[:, 16:]
    oi_ref[0] = hi[:, :16]
    oi_ref[1] = hi[:, 16:]


def _pre0(x, wt, wi):
    out_sds = jax.ShapeDtypeStruct((2, N, 16), jnp.float32)
    return pl.pallas_call(
        _pre0_body,
        grid=(N // B,),
        in_specs=[
            pl.BlockSpec((B, 128), lambda i: (i, 0)),
            pl.BlockSpec((128, 32), lambda i: (0, 0)),
            pl.BlockSpec((128, 32), lambda i: (0, 0)),
        ],
        out_specs=[
            pl.BlockSpec((2, B, 16), lambda i: (0, i, 0)),
            pl.BlockSpec((2, B, 16), lambda i: (0, i, 0)),
        ],
        out_shape=[out_sds, out_sds],
    )(x, wt, wi)


def _comb0_body(st_ref, si_ref, cnt_ref, b_ref, o_ref):
    st = jnp.concatenate([st_ref[0], st_ref[1]], axis=1)
    si = jnp.concatenate([si_ref[0], si_ref[1]], axis=1)
    cnt = jnp.maximum(cnt_ref[0][:, :1], 1.0)
    o = jnp.maximum(st + si / cnt + b_ref[...], 0.0)
    o_ref[0] = o[:, :16]
    o_ref[1] = o[:, 16:]


def _comb0(st, si, cnt, b):
    return pl.pallas_call(
        _comb0_body,
        grid=(N // B,),
        in_specs=[
            pl.BlockSpec((2, B, 16), lambda i: (0, i, 0)),
            pl.BlockSpec((2, B, 16), lambda i: (0, i, 0)),
            pl.BlockSpec((1, B, 16), lambda i: (0, i, 0)),
            pl.BlockSpec((1, 32), lambda i: (0, 0)),
        ],
        out_specs=pl.BlockSpec((2, B, 16), lambda i: (0, i, 0)),
        out_shape=jax.ShapeDtypeStruct((2, N, 16), jnp.float32),
    )(st, si, cnt, b)


def _comb_mid_body(st_ref, si_ref, cnt_ref, h_ref, wt_ref, wi_ref, b_ref, o_ref):
    st = jnp.concatenate([st_ref[0], st_ref[1]], axis=1)
    si = jnp.concatenate([si_ref[0], si_ref[1]], axis=1)
    cnt = jnp.maximum(cnt_ref[0][:, :1], 1.0)
    mean = si / cnt
    o = (jnp.dot(st, wt_ref[...], preferred_element_type=jnp.float32)
         + jnp.dot(mean, wi_ref[...], preferred_element_type=jnp.float32)
         + b_ref[...])
    o = o + jnp.concatenate([h_ref[0], h_ref[1]], axis=1)
    o = jnp.maximum(o, 0.0)
    o_ref[0] = o[:, :16]
    o_ref[1] = o[:, 16:]


def _comb_mid(st, si, cnt, h, wt, wi, b):
    return pl.pallas_call(
        _comb_mid_body,
        grid=(N // B,),
        in_specs=[
            pl.BlockSpec((2, B, 16), lambda i: (0, i, 0)),
            pl.BlockSpec((2, B, 16), lambda i: (0, i, 0)),
            pl.BlockSpec((1, B, 16), lambda i: (0, i, 0)),
            pl.BlockSpec((2, B, 16), lambda i: (0, i, 0)),
            pl.BlockSpec((32, 32), lambda i: (0, 0)),
            pl.BlockSpec((32, 32), lambda i: (0, 0)),
            pl.BlockSpec((1, 32), lambda i: (0, 0)),
        ],
        out_specs=pl.BlockSpec((2, B, 16), lambda i: (0, i, 0)),
        out_shape=jax.ShapeDtypeStruct((2, N, 16), jnp.float32),
    )(st, si, cnt, h, wt, wi, b)


def _comb_last_body(st_ref, si_ref, cnt_ref, wt_ref, wi_ref, b_ref, o_ref):
    st = jnp.concatenate([st_ref[0], st_ref[1]], axis=1)
    si = jnp.concatenate([si_ref[0], si_ref[1]], axis=1)
    cnt = jnp.maximum(cnt_ref[0][:, :1], 1.0)
    mean = si / cnt
    o = (jnp.dot(st, wt_ref[...], preferred_element_type=jnp.float32)
         + jnp.dot(mean, wi_ref[...], preferred_element_type=jnp.float32)
         + b_ref[...])
    o_ref[...] = jnp.maximum(o, 0.0)


def _comb_last(st, si, cnt, wt, wi, b):
    return pl.pallas_call(
        _comb_last_body,
        grid=(N // B,),
        in_specs=[
            pl.BlockSpec((2, B, 16), lambda i: (0, i, 0)),
            pl.BlockSpec((2, B, 16), lambda i: (0, i, 0)),
            pl.BlockSpec((1, B, 16), lambda i: (0, i, 0)),
            pl.BlockSpec((32, 64), lambda i: (0, 0)),
            pl.BlockSpec((32, 64), lambda i: (0, 0)),
            pl.BlockSpec((1, 64), lambda i: (0, 0)),
        ],
        out_specs=pl.BlockSpec((B, 64), lambda i: (i, 0)),
        out_shape=jax.ShapeDtypeStruct((N, 64), jnp.float32),
    )(st, si, cnt, wt, wi, b)


def kernel(x, edge_index_temp, edge_index_intersects, params):
    src2_t, dstg_t = _prep_edges(edge_index_temp[0], edge_index_temp[1],
                                 CHUNKS_T * 16 * J2)
    src2_i, dstg_i = _prep_edges(edge_index_intersects[0],
                                 edge_index_intersects[1], CHUNKS_I * 16 * J2)
    zeros = jnp.zeros((ROWS_PER_TILE, 16), jnp.float32)
    ones = jnp.ones((J2, G, 16), jnp.float32)

    cnt = _cnt_i(dstg_i, zeros, ones)

    p0 = params[0]
    ht0, hi0 = _pre0(x, p0["Wt"], p0["Wi"])
    st = _seg_t(ht0.reshape(2 * N, 16), src2_t, dstg_t, zeros)
    si = _seg_i(hi0.reshape(2 * N, 16), src2_i, dstg_i, zeros)
    h = _comb0(st, si, cnt, (p0["bt"] + p0["bi"]).reshape(1, 32))

    for p in params[1:4]:
        tab = h.reshape(2 * N, 16)
        st = _seg_t(tab, src2_t, dstg_t, zeros)
        si = _seg_i(tab, src2_i, dstg_i, zeros)
        h = _comb_mid(st, si, cnt, h, p["Wt"], p["Wi"],
                      (p["bt"] + p["bi"]).reshape(1, 32))

    p4 = params[4]
    tab = h.reshape(2 * N, 16)
    st = _seg_t(tab, src2_t, dstg_t, zeros)
    si = _seg_i(tab, src2_i, dstg_i, zeros)
    return _comb_last(st, si, cnt, p4["Wt"], p4["Wi"],
                      (p4["bt"] + p4["bi"]).reshape(1, 64))


# speedup vs baseline: 1.0064x; 1.0064x over previous
import functools

import jax
import jax.numpy as jnp
from jax import lax
from jax.experimental import pallas as pl
from jax.experimental.pallas import tpu as pltpu
from jax.experimental.pallas import tpu_sc as plsc

N = 100000
NPAD = 100096
ROWS_PER_TILE = 6256
TRASH = 100000
J2 = 2
G = 384
B = 2000

CHUNKS_T = 10
CHUNKS_I = 124


_MESH = plsc.VectorSubcoreMesh(core_axis_name="c", subcore_axis_name="s")


def _gather_waits(table, src_v, rows_v, sem_g):
    for j in range(J2):
        pltpu.make_async_copy(table.at[src_v.at[j]], rows_v.at[j], sem_g).wait()


def _gather_fires(table, src_v, rows_v, sem_g):
    for j in range(J2):
        pltpu.async_copy(table.at[src_v.at[j]], rows_v.at[j], sem_g)


def _scatter_waits(rows_v, acc, dst_v, sem_s):
    for j in range(J2):
        pltpu.make_async_copy(rows_v.at[j], acc.at[dst_v.at[j]], sem_s).wait()


def _scatter_fires(rows_v, acc, dst_v, sem_s):
    for j in range(J2):
        pltpu.async_copy(rows_v.at[j], acc.at[dst_v.at[j]], sem_s, add=True)


def _fill_trash(dst_v):
    tr = jnp.full((16,), TRASH, jnp.int32)
    for j in range(J2):
        for i in range(G // 16):
            dst_v[j, pl.ds(i * 16, 16)] = tr


def _seg_phase(ch, table, src2, dstg, out, c, s,
               src0, dst0, rows0, src1, dst1, rows1, acc, zeros, sem_g, sem_s):
    pltpu.sync_copy(zeros, acc.at[pl.ds(s * ROWS_PER_TILE, ROWS_PER_TILE)])
    _fill_trash(dst1)
    plsc.subcore_barrier()
    _scatter_fires(rows1, acc, dst1, sem_s)
    base0 = s * ch * J2
    pltpu.sync_copy(src2.at[c].at[pl.ds(base0, J2)], src0)
    pltpu.sync_copy(dstg.at[pl.ds(base0, J2)], dst0)
    _gather_fires(table, src0, rows0, sem_g)

    def half(bsrc, bdst, brows, osrc, odst, orows, k, nxt):
        _gather_waits(table, bsrc, brows, sem_g)
        _scatter_waits(orows, acc, odst, sem_s)
        _scatter_fires(brows, acc, bdst, sem_s)
        base = (s * ch + nxt) * J2
        pltpu.sync_copy(src2.at[c].at[pl.ds(base, J2)], osrc)
        pltpu.sync_copy(dstg.at[pl.ds(base, J2)], odst)
        _gather_fires(table, osrc, orows, sem_g)

    def pair(k, carry):
        half(src0, dst0, rows0, src1, dst1, rows1, k, 2 * k + 1)
        half(src1, dst1, rows1, src0, dst0, rows0, k, 2 * k + 2)
        return carry

    lax.fori_loop(0, ch // 2, pair, 0)
    _gather_waits(table, src0, rows0, sem_g)
    _scatter_waits(rows1, acc, dst1, sem_s)
    plsc.subcore_barrier()
    pltpu.sync_copy(
        acc.at[pl.ds(s * ROWS_PER_TILE, ROWS_PER_TILE)],
        out.at[c].at[pl.ds(s * ROWS_PER_TILE, ROWS_PER_TILE)],
    )


def _cnt_phase(ch, dstg, out, c, s, dst0, dst1, rows1, acc, zeros, ones, sem_s):
    pltpu.sync_copy(zeros, acc.at[pl.ds(s * ROWS_PER_TILE, ROWS_PER_TILE)])
    pltpu.sync_copy(ones, rows1)
    _fill_trash(dst1)
    plsc.subcore_barrier()
    _scatter_fires(rows1, acc, dst1, sem_s)
    base0 = s * ch * J2
    pltpu.sync_copy(dstg.at[pl.ds(base0, J2)], dst0)

    def half(bdst, odst, nxt):
        _scatter_waits(rows1, acc, odst, sem_s)
        _scatter_fires(rows1, acc, bdst, sem_s)
        base = (s * ch + nxt) * J2
        pltpu.sync_copy(dstg.at[pl.ds(base, J2)], odst)

    def pair(k, carry):
        half(dst0, dst1, 2 * k + 1)
        half(dst1, dst0, 2 * k + 2)
        return carry

    lax.fori_loop(0, ch // 2, pair, 0)
    _scatter_waits(rows1, acc, dst1, sem_s)
    plsc.subcore_barrier()
    pltpu.sync_copy(
        acc.at[pl.ds(s * ROWS_PER_TILE, ROWS_PER_TILE)],
        out.at[c].at[pl.ds(s * ROWS_PER_TILE, ROWS_PER_TILE)],
    )


_SC_SCRATCH = [
    pltpu.VMEM((J2, G), jnp.int32),
    pltpu.VMEM((J2, G), jnp.int32),
    pltpu.VMEM((J2, G, 16), jnp.float32),
    pltpu.VMEM((J2, G), jnp.int32),
    pltpu.VMEM((J2, G), jnp.int32),
    pltpu.VMEM((J2, G, 16), jnp.float32),
    pltpu.VMEM_SHARED((NPAD, 16), jnp.float32),
    pltpu.SemaphoreType.DMA,
    pltpu.SemaphoreType.DMA,
]

_SDS = jax.ShapeDtypeStruct((2, NPAD, 16), jnp.float32)


@functools.partial(
    pl.kernel,
    out_type=(_SDS, _SDS, _SDS),
    mesh=_MESH,
    scratch_types=_SC_SCRATCH,
    compiler_params=pltpu.CompilerParams(use_tc_tiling_on_sc=False),
)
def _layer0_sc(tab_t, tab_i, src2_t, dstg_t, src2_i, dstg_i, zeros, ones,
               st, si, cnt,
               src0, dst0, rows0, src1, dst1, rows1, acc, sem_g, sem_s):
    c = lax.axis_index("c")
    s = lax.axis_index("s")
    _seg_phase(CHUNKS_T, tab_t, src2_t, dstg_t, st, c, s,
               src0, dst0, rows0, src1, dst1, rows1, acc, zeros, sem_g, sem_s)
    _seg_phase(CHUNKS_I, tab_i, src2_i, dstg_i, si, c, s,
               src0, dst0, rows0, src1, dst1, rows1, acc, zeros, sem_g, sem_s)
    _cnt_phase(CHUNKS_I, dstg_i, cnt, c, s,
               dst0, dst1, rows1, acc, zeros, ones, sem_s)


@functools.partial(
    pl.kernel,
    out_type=(_SDS, _SDS),
    mesh=_MESH,
    scratch_types=_SC_SCRATCH,
    compiler_params=pltpu.CompilerParams(use_tc_tiling_on_sc=False),
)
def _layer_sc(tab, src2_t, dstg_t, src2_i, dstg_i, zeros,
              st, si,
              src0, dst0, rows0, src1, dst1, rows1, acc, sem_g, sem_s):
    c = lax.axis_index("c")
    s = lax.axis_index("s")
    _seg_phase(CHUNKS_T, tab, src2_t, dstg_t, st, c, s,
               src0, dst0, rows0, src1, dst1, rows1, acc, zeros, sem_g, sem_s)
    _seg_phase(CHUNKS_I, tab, src2_i, dstg_i, si, c, s,
               src0, dst0, rows0, src1, dst1, rows1, acc, zeros, sem_g, sem_s)


def _prep_edges(src, dst, ngroups):
    e = src.shape[0]
    epad = (ngroups + J2) * G
    srcp = jnp.concatenate([src, jnp.zeros((epad - e,), jnp.int32)])
    dstp = jnp.concatenate([dst, jnp.full((epad - e,), TRASH, jnp.int32)])
    src2 = jnp.stack([srcp, srcp + N]).reshape(2, ngroups + J2, G)
    return src2, dstp.reshape(ngroups + J2, G)


def _pre0_body(x_ref, wt_ref, wi_ref, ot_ref, oi_ref):
    xb = x_ref[...]
    ht = jnp.dot(xb, wt_ref[...], preferred_element_type=jnp.float32)
    hi = jnp.dot(xb, wi_ref[...], preferred_element_type=jnp.float32)
    ot_ref[0] = ht[:, :16]
    ot_ref[1] = ht[:, 16:]
    oi_ref[0] = hi[:, :16]
    oi_ref[1] = hi[:, 16:]


def _pre0(x, wt, wi):
    out_sds = jax.ShapeDtypeStruct((2, N, 16), jnp.float32)
    return pl.pallas_call(
        _pre0_body,
        grid=(N // B,),
        in_specs=[
            pl.BlockSpec((B, 128), lambda i: (i, 0)),
            pl.BlockSpec((128, 32), lambda i: (0, 0)),
            pl.BlockSpec((128, 32), lambda i: (0, 0)),
        ],
        out_specs=[
            pl.BlockSpec((2, B, 16), lambda i: (0, i, 0)),
            pl.BlockSpec((2, B, 16), lambda i: (0, i, 0)),
        ],
        out_shape=[out_sds, out_sds],
    )(x, wt, wi)


def _comb0_body(st_ref, si_ref, cnt_ref, b_ref, o_ref):
    st = jnp.concatenate([st_ref[0], st_ref[1]], axis=1)
    si = jnp.concatenate([si_ref[0], si_ref[1]], axis=1)
    cnt = jnp.maximum(cnt_ref[0][:, :1], 1.0)
    o = jnp.maximum(st + si / cnt + b_ref[...], 0.0)
    o_ref[0] = o[:, :16]
    o_ref[1] = o[:, 16:]


def _comb0(st, si, cnt, b):
    return pl.pallas_call(
        _comb0_body,
        grid=(N // B,),
        in_specs=[
            pl.BlockSpec((2, B, 16), lambda i: (0, i, 0)),
            pl.BlockSpec((2, B, 16), lambda i: (0, i, 0)),
            pl.BlockSpec((1, B, 16), lambda i: (0, i, 0)),
            pl.BlockSpec((1, 32), lambda i: (0, 0)),
        ],
        out_specs=pl.BlockSpec((2, B, 16), lambda i: (0, i, 0)),
        out_shape=jax.ShapeDtypeStruct((2, N, 16), jnp.float32),
    )(st, si, cnt, b)


def _comb_mid_body(st_ref, si_ref, cnt_ref, h_ref, wt_ref, wi_ref, b_ref, o_ref):
    st = jnp.concatenate([st_ref[0], st_ref[1]], axis=1)
    si = jnp.concatenate([si_ref[0], si_ref[1]], axis=1)
    cnt = jnp.maximum(cnt_ref[0][:, :1], 1.0)
    mean = si / cnt
    o = (jnp.dot(st, wt_ref[...], preferred_element_type=jnp.float32)
         + jnp.dot(mean, wi_ref[...], preferred_element_type=jnp.float32)
         + b_ref[...])
    o = o + jnp.concatenate([h_ref[0], h_ref[1]], axis=1)
    o = jnp.maximum(o, 0.0)
    o_ref[0] = o[:, :16]
    o_ref[1] = o[:, 16:]


def _comb_mid(st, si, cnt, h, wt, wi, b):
    return pl.pallas_call(
        _comb_mid_body,
        grid=(N // B,),
        in_specs=[
            pl.BlockSpec((2, B, 16), lambda i: (0, i, 0)),
            pl.BlockSpec((2, B, 16), lambda i: (0, i, 0)),
            pl.BlockSpec((1, B, 16), lambda i: (0, i, 0)),
            pl.BlockSpec((2, B, 16), lambda i: (0, i, 0)),
            pl.BlockSpec((32, 32), lambda i: (0, 0)),
            pl.BlockSpec((32, 32), lambda i: (0, 0)),
            pl.BlockSpec((1, 32), lambda i: (0, 0)),
        ],
        out_specs=pl.BlockSpec((2, B, 16), lambda i: (0, i, 0)),
        out_shape=jax.ShapeDtypeStruct((2, N, 16), jnp.float32),
    )(st, si, cnt, h, wt, wi, b)


def _comb_last_body(st_ref, si_ref, cnt_ref, wt_ref, wi_ref, b_ref, o_ref):
    st = jnp.concatenate([st_ref[0], st_ref[1]], axis=1)
    si = jnp.concatenate([si_ref[0], si_ref[1]], axis=1)
    cnt = jnp.maximum(cnt_ref[0][:, :1], 1.0)
    mean = si / cnt
    o = (jnp.dot(st, wt_ref[...], preferred_element_type=jnp.float32)
         + jnp.dot(mean, wi_ref[...], preferred_element_type=jnp.float32)
         + b_ref[...])
    o_ref[...] = jnp.maximum(o, 0.0)


def _comb_last(st, si, cnt, wt, wi, b):
    return pl.pallas_call(
        _comb_last_body,
        grid=(N // B,),
        in_specs=[
            pl.BlockSpec((2, B, 16), lambda i: (0, i, 0)),
            pl.BlockSpec((2, B, 16), lambda i: (0, i, 0)),
            pl.BlockSpec((1, B, 16), lambda i: (0, i, 0)),
            pl.BlockSpec((32, 64), lambda i: (0, 0)),
            pl.BlockSpec((32, 64), lambda i: (0, 0)),
            pl.BlockSpec((1, 64), lambda i: (0, 0)),
        ],
        out_specs=pl.BlockSpec((B, 64), lambda i: (i, 0)),
        out_shape=jax.ShapeDtypeStruct((N, 64), jnp.float32),
    )(st, si, cnt, wt, wi, b)


def kernel(x, edge_index_temp, edge_index_intersects, params):
    src2_t, dstg_t = _prep_edges(edge_index_temp[0], edge_index_temp[1],
                                 CHUNKS_T * 16 * J2)
    src2_i, dstg_i = _prep_edges(edge_index_intersects[0],
                                 edge_index_intersects[1], CHUNKS_I * 16 * J2)
    zeros = jnp.zeros((ROWS_PER_TILE, 16), jnp.float32)
    ones = jnp.ones((J2, G, 16), jnp.float32)

    p0 = params[0]
    ht0, hi0 = _pre0(x, p0["Wt"], p0["Wi"])
    st, si, cnt = _layer0_sc(ht0.reshape(2 * N, 16), hi0.reshape(2 * N, 16),
                             src2_t, dstg_t, src2_i, dstg_i, zeros, ones)
    h = _comb0(st, si, cnt, (p0["bt"] + p0["bi"]).reshape(1, 32))

    for p in params[1:4]:
        st, si = _layer_sc(h.reshape(2 * N, 16), src2_t, dstg_t,
                           src2_i, dstg_i, zeros)
        h = _comb_mid(st, si, cnt, h, p["Wt"], p["Wi"],
                      (p["bt"] + p["bi"]).reshape(1, 32))

    p4 = params[4]
    st, si = _layer_sc(h.reshape(2 * N, 16), src2_t, dstg_t,
                       src2_i, dstg_i, zeros)
    return _comb_last(st, si, cnt, p4["Wt"], p4["Wi"],
                      (p4["bt"] + p4["bi"]).reshape(1, 64))
